# baseline (speedup 1.0000x reference)
import functools
import math

import jax
import jax.numpy as jnp
from jax import lax
from jax.experimental import pallas as pl
from jax.experimental.pallas import tpu as pltpu
from jax.experimental.pallas import tpu_sc as plsc
from jax.experimental.layout import Layout, with_layout_constraint

NC = 2
NS = 16
NW = NC * NS
CH = 128
NR = 4
PADW = 2


def _make_lookup(T, Btok, V, D, scale):
    n_ch = T
    assert Btok == NW * CH
    assert n_ch % NR == 0
    mesh = plsc.VectorSubcoreMesh(
        core_axis_name="c", subcore_axis_name="s",
        num_cores=NC, num_subcores=NS)

    @functools.partial(
        pl.kernel,
        out_type=jax.ShapeDtypeStruct((T, Btok, PADW, D), jnp.float32),
        mesh=mesh,
        scratch_types=[
            pltpu.VMEM((T, CH), jnp.int32),
            pltpu.VMEM((NR, CH, D), jnp.float32),
            pltpu.VMEM((2, CH, PADW, D), jnp.float32),
            pltpu.SemaphoreType.DMA((NR,)),
            pltpu.SemaphoreType.DMA((2,)),
        ],
        compiler_params=pltpu.CompilerParams(use_tc_tiling_on_sc=False),
    )
    def lookup(tokR, table_hbm, out_hbm, idx_v, rows_v, stg_v, gsem, osem):
        wid = lax.axis_index("s") * NC + lax.axis_index("c")
        b0 = wid * CH

        pltpu.sync_copy(tokR.at[wid], idx_v)

        def gather(t, rb):
            return pltpu.make_async_copy(
                table_hbm.at[idx_v.at[t]], rows_v.at[rb], gsem.at[rb])

        def out_copy(t, sb):
            return pltpu.make_async_copy(
                stg_v.at[sb], out_hbm.at[t, pl.ds(b0, CH)], osem.at[sb])

        for rb in range(NR):
            gather(rb, rb).start()

        def do_chunk(t, rb, sb, refill):
            gather(t, rb).wait()

            @pl.when(t >= 2)
            def _():
                out_copy(t - 2, sb).wait()

            @plsc.parallel_loop(0, CH, unroll=8)
            def _scale(rr):
                for c in range(D // 16):
                    sl = pl.ds(c * 16, 16)
                    stg_v[sb, rr, 0, sl] = rows_v[rb, rr, sl] * scale

            out_copy(t, sb).start()
            if refill:
                gather(t + NR, rb).start()

        n_outer = n_ch // NR

        @pl.loop(0, n_outer - 1)
        def _main(step):
            for j in range(NR):
                t = step * NR + j
                do_chunk(t, j, j % 2, refill=True)

        for j in range(NR):
            t = (n_outer - 1) * NR + j
            do_chunk(t, j, j % 2, refill=False)

        out_copy(n_ch - 2, (n_ch - 2) % 2).wait()
        out_copy(n_ch - 1, (n_ch - 1) % 2).wait()

    return lookup


def kernel(tokens, table):
    Btok, T = tokens.shape
    V, D = table.shape
    assert Btok == NW * CH and D % 16 == 0
    scale = math.sqrt(D)

    tokR = tokens.T.astype(jnp.int32).reshape(T, NW, CH).transpose(1, 0, 2)
    tbl = with_layout_constraint(table, Layout((0, 1), tiling=((8, 64),)))
    out4 = _make_lookup(T, Btok, V, D, scale)(tokR, tbl)
    return out4[:, :, 0, :].transpose(1, 0, 2)


# speedup vs baseline: 3.1361x; 3.1361x over previous
import functools
import math

import jax
import jax.numpy as jnp
from jax import lax
from jax.experimental import pallas as pl
from jax.experimental.pallas import tpu as pltpu
from jax.experimental.pallas import tpu_sc as plsc
from jax.experimental.layout import Layout, with_layout_constraint

NC = 2
NS = 16
NW = NC * NS
CH = 128
NR = 4
PADW = 2


def _make_lookup(T, Btok, V, D, scale):
    n_ch = T
    assert Btok == NW * CH
    assert n_ch % NR == 0
    mesh = plsc.VectorSubcoreMesh(
        core_axis_name="c", subcore_axis_name="s",
        num_cores=NC, num_subcores=NS)

    @functools.partial(
        pl.kernel,
        out_type=jax.ShapeDtypeStruct((T, Btok, D), jnp.float32),
        mesh=mesh,
        scratch_types=[
            pltpu.VMEM((T, CH), jnp.int32),
            pltpu.VMEM((NR, CH, D), jnp.float32),
            pltpu.SemaphoreType.DMA((NR,)),
            pltpu.SemaphoreType.DMA((NR,)),
        ],
        compiler_params=pltpu.CompilerParams(use_tc_tiling_on_sc=False),
    )
    def lookup(tokR, table_hbm, out_hbm, idx_v, rows_v, gsem, osem):
        wid = lax.axis_index("s") * NC + lax.axis_index("c")
        b0 = wid * CH

        pltpu.sync_copy(tokR.at[wid], idx_v)

        def gather(t, rb):
            return pltpu.make_async_copy(
                table_hbm.at[idx_v.at[t]], rows_v.at[rb], gsem.at[rb])

        def out_copy(t, rb):
            return pltpu.make_async_copy(
                rows_v.at[rb], out_hbm.at[t, pl.ds(b0, CH)], osem.at[rb])

        for rb in range(NR):
            gather(rb, rb).start()

        def do_chunk(t, rb, refill):
            gather(t, rb).wait()

            @plsc.parallel_loop(0, CH, unroll=8)
            def _scale(rr):
                for c in range(D // 16):
                    sl = pl.ds(c * 16, 16)
                    rows_v[rb, rr, sl] = rows_v[rb, rr, sl] * scale

            cp = out_copy(t, rb)
            cp.start()
            cp.wait()
            if refill:
                gather(t + NR, rb).start()

        n_outer = n_ch // NR

        @pl.loop(0, n_outer - 1)
        def _main(step):
            for j in range(NR):
                do_chunk(step * NR + j, j, refill=True)

        for j in range(NR):
            do_chunk((n_outer - 1) * NR + j, j, refill=False)

    return lookup


def kernel(tokens, table):
    Btok, T = tokens.shape
    V, D = table.shape
    assert Btok == NW * CH and D % 16 == 0
    scale = math.sqrt(D)

    tokR = tokens.T.astype(jnp.int32).reshape(T, NW, CH).transpose(1, 0, 2)
    tbl = with_layout_constraint(table, Layout((0, 1), tiling=((8, 64),)))
    out3 = _make_lookup(T, Btok, V, D, scale)(tokR, tbl)
    return out3.transpose(1, 0, 2)
